# Initial kernel scaffold; baseline (speedup 1.0000x reference)
import jax
import jax.numpy as jnp
from jax.experimental import pallas as pl


def kernel(pts, feats, pool_ids1, edge_index1, pool_ids2, edge_index2, batch_ids_out, k1a_11, k1b_11, w_11, bn_g1, bn_b1, k1a_12, k1b_12, w_12, bn_g2, bn_b2, k1a_21, k1b_21, w_21, bn_g3, bn_b3, k1a_22, k1b_22, w_22, bn_g4, bn_b4, fc1_w, fc1_b, bn_g5, bn_b5, fc2_w, fc2_b):
    raise NotImplementedError("write your pallas kernel here")


# speedup vs baseline: 1.0034x; 1.0034x over previous
import functools

import jax
import jax.numpy as jnp
from jax import lax
from jax.experimental import pallas as pl
from jax.experimental.pallas import tpu as pltpu

N = 50000
M1 = 12500
M2 = 3125
E1 = 400000
E2 = 100000
B = 8


def _segment_mean(data, seg, num):
    s = jax.ops.segment_sum(data, seg, num_segments=num)
    c = jax.ops.segment_sum(jnp.ones((data.shape[0],), jnp.float32), seg, num_segments=num)
    return s / jnp.maximum(c, 1.0)[:, None]


def _bn(x, g, b):
    m = jnp.mean(x, axis=0)
    v = jnp.var(x, axis=0)
    return (x - m) / jnp.sqrt(v + 1e-5) * g + b


def _mcconv(pts, feats, edges, ka, kb, w, num, radius):
    src = edges[0]
    dst = edges[1]
    rel = (pts[dst] - pts[src]) / radius
    h = jax.nn.relu(rel @ ka)
    we = h @ kb
    counts = jax.ops.segment_sum(jnp.ones(src.shape, jnp.float32), src, num_segments=num)
    pdf = counts / jnp.maximum(jnp.mean(counts), 1e-6)
    msg = feats[src] * we / jnp.maximum(pdf[src], 1e-3)[:, None]
    agg = jax.ops.segment_sum(msg, dst, num_segments=num)
    deg = jax.ops.segment_sum(jnp.ones(dst.shape, jnp.float32), dst, num_segments=num)
    agg = agg / jnp.maximum(deg, 1.0)[:, None]
    return agg @ w


def _head_body(f_ref, ids_ref, g4_ref, b4_ref, w1_ref, c1_ref, g5_ref, b5_ref,
               w2_ref, c2_ref, out_ref):
    f = f_ref[...]
    ids = ids_ref[...]
    seg = lax.broadcasted_iota(jnp.int32, (B, M2), 0)
    onehot = (ids == seg).astype(jnp.float32)
    cnt = jnp.sum(onehot, axis=1, keepdims=True)
    sums = jnp.dot(onehot, f, preferred_element_type=jnp.float32)
    gm = sums / jnp.maximum(cnt, 1.0)
    m = jnp.mean(gm, axis=0)
    v = jnp.mean((gm - m) ** 2, axis=0)
    gm = jax.nn.relu((gm - m) / jnp.sqrt(v + 1e-5) * g4_ref[...] + b4_ref[...])
    gm = jnp.dot(gm, w1_ref[...], preferred_element_type=jnp.float32) + c1_ref[...]
    m = jnp.mean(gm, axis=0)
    v = jnp.mean((gm - m) ** 2, axis=0)
    gm = jax.nn.relu((gm - m) / jnp.sqrt(v + 1e-5) * g5_ref[...] + b5_ref[...])
    out_ref[...] = jnp.dot(gm, w2_ref[...], preferred_element_type=jnp.float32) + c2_ref[...]


def _head(f, batch_ids, g4, b4, w1, c1, g5, b5, w2, c2):
    return pl.pallas_call(
        _head_body,
        out_shape=jax.ShapeDtypeStruct((B, w2.shape[1]), jnp.float32),
    )(f, batch_ids.reshape(1, M2).astype(jnp.int32), g4, b4, w1, c1, g5, b5, w2, c2)


def kernel(pts, feats, pool_ids1, edge_index1, pool_ids2, edge_index2, batch_ids_out,
           k1a_11, k1b_11, w_11, bn_g1, bn_b1, k1a_12, k1b_12, w_12, bn_g2, bn_b2,
           k1a_21, k1b_21, w_21, bn_g3, bn_b3, k1a_22, k1b_22, w_22, bn_g4, bn_b4,
           fc1_w, fc1_b, bn_g5, bn_b5, fc2_w, fc2_b):
    pts1 = _segment_mean(pts, pool_ids1, M1)
    f = _segment_mean(feats, pool_ids1, M1)
    f = _mcconv(pts1, f, edge_index1, k1a_11, k1b_11, w_11, M1, 0.1)
    f = jax.nn.relu(_bn(f, bn_g1, bn_b1))
    f = _mcconv(pts1, f, edge_index1, k1a_12, k1b_12, w_12, M1, 0.1)
    pts2 = _segment_mean(pts1, pool_ids2, M2)
    f = _segment_mean(f, pool_ids2, M2)
    f = jax.nn.relu(_bn(f, bn_g2, bn_b2))
    f = _mcconv(pts2, f, edge_index2, k1a_21, k1b_21, w_21, M2, 0.2)
    f = jax.nn.relu(_bn(f, bn_g3, bn_b3))
    f = _mcconv(pts2, f, edge_index2, k1a_22, k1b_22, w_22, M2, 0.2)
    return _head(f, batch_ids_out, bn_g4, bn_b4, fc1_w, fc1_b, bn_g5, bn_b5, fc2_w, fc2_b)


# speedup vs baseline: 3.1936x; 3.1828x over previous
import jax
import jax.numpy as jnp
from jax import lax
from jax.experimental import pallas as pl
from jax.experimental.pallas import tpu as pltpu
from jax.experimental.pallas import tpu_sc as plsc

N = 50000
NP = 51200
M1 = 12500
M1P = 12800
H1 = 6400
M2 = 3125
M2P = 3584
H2 = 1792
E1 = 400000
E1P = 401408
E2 = 100000
E2P = 106496
NB = 8
CHE = 128
CHP = 80

_F32 = jnp.float32
_I32 = jnp.int32


def _mesh():
    return plsc.VectorSubcoreMesh(core_axis_name="c", subcore_axis_name="s")


def _zero16():
    return jnp.zeros((16,), _F32)


def _zero_rows(ref, nrows, ncols):
    nb = ncols // 16

    def body(i, _):
        for b in range(nb):
            ref[i, pl.ds(b * 16, 16)] = _zero16()
        return _

    lax.fori_loop(0, nrows, body, None)


def _zero_vec(ref, n):
    def body(i, _):
        ref[pl.ds(i * 16, 16)] = _zero16()
        return _

    lax.fori_loop(0, n // 16, body, None)


def _fill_ones(ref, n):
    one = jnp.ones((16,), _F32)

    def body(i, _):
        ref[pl.ds(i * 16, 16)] = one
        return _

    lax.fori_loop(0, n // 16, body, None)


def _localize(idx_ref, n, lo, h):
    def body(q, _):
        sl = pl.ds(q * 16, 16)
        v = idx_ref[sl] - lo
        ok = (v >= 0) & (v < h)
        idx_ref[sl] = jnp.where(ok, v, h)
        return _

    lax.fori_loop(0, n // 16, body, None)


def _a1_body(rows16, ids, src, dst, pxc, pyc, pzc, onec,
             table16, px1, py1, pz1,
             slab, ones_buf, zb, pxb, pyb, pzb, cntb, degb,
             idx0, idx1, rowb0, rowb1, cb0, cb1,
             hia0, hia1, hib0, hib1,
             pld0, pld1, psc0, psc1, hld0, hld1, hsc0, hsc1,
             sh_table, sh_cnt, sh_deg, sh_px, sh_py, sh_pz, sh_one):
    c = lax.axis_index("c")
    s = lax.axis_index("s")
    lo = c * H1
    vecs = (sh_cnt, sh_deg, sh_px, sh_py, sh_pz, sh_one)

    _zero_rows(slab, 400, 16)
    pltpu.sync_copy(slab, sh_table.at[pl.ds(s * 400, 400)])
    _zero_vec(zb, 400)
    for t in vecs:
        pltpu.sync_copy(zb, t.at[pl.ds(s * 400, 400)])

    @pl.when(s == 0)
    def _():
        pltpu.sync_copy(slab.at[pl.ds(0, 16)], sh_table.at[pl.ds(H1, 16)])
        for t in vecs:
            pltpu.sync_copy(zb.at[pl.ds(0, 16)], t.at[pl.ds(H1, 16)])

    plsc.subcore_barrier()

    pslots = ((idx0, rowb0, cb0, pld0, psc0), (idx1, rowb1, cb1, pld1, psc1))

    def pdrain(rowb, cb, sc):
        pltpu.make_async_copy(rows16.at[pl.ds(0, CHP)], rowb, sc).wait()
        for q in range(4):
            pltpu.make_async_copy(pxc.at[pl.ds(0, CHP)], cb.at[q], sc).wait()

    def pbody(k2, _):
        descs = {}
        for p, (idx, rowb, cb, ld, sc) in enumerate(pslots):
            kk = 2 * k2 + p
            off = s * 3200 + kk * CHP

            @pl.when(k2 > 0)
            def _():
                pdrain(rowb, cb, sc)

            dl = [pltpu.async_copy(ids.at[pl.ds(off, CHP)], idx, ld),
                  pltpu.async_copy(rows16.at[pl.ds(off, CHP)], rowb, ld)]
            for q, colsrc in enumerate((pxc, pyc, pzc, onec)):
                dl.append(pltpu.async_copy(colsrc.at[pl.ds(off, CHP)],
                                           cb.at[q], ld))
            descs[p] = dl
        for p, (idx, rowb, cb, ld, sc) in enumerate(pslots):
            for dd in descs[p]:
                dd.wait()
            _localize(idx, CHP, lo, H1)
            pltpu.async_copy(rowb, sh_table.at[idx], sc, add=True)
            for q, t in enumerate((sh_px, sh_py, sh_pz, sh_one)):
                pltpu.async_copy(cb.at[q], t.at[idx], sc, add=True)
        return _

    lax.fori_loop(0, 20, pbody, None)
    for idx, rowb, cb, ld, sc in pslots:
        pdrain(rowb, cb, sc)

    _fill_ones(ones_buf, CHE)
    nk2 = (E1P // CHE // 16) // 2
    hslots = ((hia0, hib0, hld0, hsc0), (hia1, hib1, hld1, hsc1))

    def hdrain(sc):
        for _q in range(2):
            pltpu.make_async_copy(onec.at[pl.ds(0, CHE)], ones_buf, sc).wait()

    def hbody(k2, _):
        descs = {}
        for p, (hia, hib, ld, sc) in enumerate(hslots):
            kk = 2 * k2 + p
            off = (s + 16 * kk) * CHE

            @pl.when(k2 > 0)
            def _():
                hdrain(sc)

            descs[p] = [pltpu.async_copy(src.at[pl.ds(off, CHE)], hia, ld),
                        pltpu.async_copy(dst.at[pl.ds(off, CHE)], hib, ld)]
        for p, (hia, hib, ld, sc) in enumerate(hslots):
            for dd in descs[p]:
                dd.wait()
            _localize(hia, CHE, lo, H1)
            _localize(hib, CHE, lo, H1)
            pltpu.async_copy(ones_buf, sh_cnt.at[hia], sc, add=True)
            pltpu.async_copy(ones_buf, sh_deg.at[hib], sc, add=True)
        return _

    lax.fori_loop(0, nk2, hbody, None)
    for hia, hib, ld, sc in hslots:
        hdrain(sc)

    plsc.subcore_barrier()

    base = s * 400
    glob = c * H1 + s * 400
    ci = lax.broadcasted_iota(_I32, (16,), 0)

    pltpu.sync_copy(sh_table.at[pl.ds(base, 400)], slab)
    pltpu.sync_copy(sh_cnt.at[pl.ds(base, 400)], cntb)
    pltpu.sync_copy(sh_deg.at[pl.ds(base, 400)], degb)

    def drow16(q, _):
        cnt16 = cntb[pl.ds(q * 16, 16)]
        deg16 = degb[pl.ds(q * 16, 16)]
        for r16 in range(16):
            r = q * 16 + r16
            rowv = slab[r, pl.ds(0, 16)]
            invv = 1.0 / jnp.maximum(jnp.broadcast_to(rowv[6], (16,)), 1.0)
            out = rowv * invv
            out = jnp.where(ci == 7, jnp.broadcast_to(cnt16[r16], (16,)), out)
            out = jnp.where(ci == 8, jnp.broadcast_to(deg16[r16], (16,)), out)
            slab[r, pl.ds(0, 16)] = out
        return _

    lax.fori_loop(0, 25, drow16, None)
    pltpu.sync_copy(slab, table16.at[pl.ds(glob, 400)])

    pltpu.sync_copy(sh_px.at[pl.ds(base, 400)], pxb)
    pltpu.sync_copy(sh_py.at[pl.ds(base, 400)], pyb)
    pltpu.sync_copy(sh_pz.at[pl.ds(base, 400)], pzb)
    pltpu.sync_copy(sh_one.at[pl.ds(base, 400)], zb)

    def dv(q, _):
        sl = pl.ds(q * 16, 16)
        inv = 1.0 / jnp.maximum(zb[sl], 1.0)
        pxb[sl] = pxb[sl] * inv
        pyb[sl] = pyb[sl] * inv
        pzb[sl] = pzb[sl] * inv
        return _

    lax.fori_loop(0, 25, dv, None)

    pltpu.sync_copy(pxb, px1.at[pl.ds(glob, 400)])
    pltpu.sync_copy(pyb, py1.at[pl.ds(glob, 400)])
    pltpu.sync_copy(pzb, pz1.at[pl.ds(glob, 400)])


def _a1(rows16, ids, src, dst, pxc, pyc, pzc, onec):
    f = pl.kernel(
        _a1_body,
        out_type=[
            jax.ShapeDtypeStruct((M1P, 16), _F32),
            jax.ShapeDtypeStruct((M1P,), _F32),
            jax.ShapeDtypeStruct((M1P,), _F32),
            jax.ShapeDtypeStruct((M1P,), _F32),
        ],
        mesh=_mesh(),
        compiler_params=pltpu.CompilerParams(use_tc_tiling_on_sc=False),
        scratch_types=[
            pltpu.VMEM((400, 16), _F32),
            pltpu.VMEM((CHE,), _F32),
            pltpu.VMEM((400,), _F32),
            pltpu.VMEM((400,), _F32),
            pltpu.VMEM((400,), _F32),
            pltpu.VMEM((400,), _F32),
            pltpu.VMEM((400,), _F32),
            pltpu.VMEM((400,), _F32),
            pltpu.VMEM((CHP,), _I32),
            pltpu.VMEM((CHP,), _I32),
            pltpu.VMEM((CHP, 16), _F32),
            pltpu.VMEM((CHP, 16), _F32),
            pltpu.VMEM((4, CHP), _F32),
            pltpu.VMEM((4, CHP), _F32),
            pltpu.VMEM((CHE,), _I32),
            pltpu.VMEM((CHE,), _I32),
            pltpu.VMEM((CHE,), _I32),
            pltpu.VMEM((CHE,), _I32),
            pltpu.SemaphoreType.DMA,
            pltpu.SemaphoreType.DMA,
            pltpu.SemaphoreType.DMA,
            pltpu.SemaphoreType.DMA,
            pltpu.SemaphoreType.DMA,
            pltpu.SemaphoreType.DMA,
            pltpu.SemaphoreType.DMA,
            pltpu.SemaphoreType.DMA,
            pltpu.VMEM_SHARED((H1 + 16, 16), _F32),
            pltpu.VMEM_SHARED((H1 + 16,), _F32),
            pltpu.VMEM_SHARED((H1 + 16,), _F32),
            pltpu.VMEM_SHARED((H1 + 16,), _F32),
            pltpu.VMEM_SHARED((H1 + 16,), _F32),
            pltpu.VMEM_SHARED((H1 + 16,), _F32),
            pltpu.VMEM_SHARED((H1 + 16,), _F32),
        ],
    )
    return f(rows16, ids, src, dst, pxc, pyc, pzc, onec)


def _a2_body(f12p, pids2, px1, py1, pz1, src, dst,
             fsum2, pone, table2,
             ones_buf, zb, pxb, pyb, pzb, cntb, degb, tbuf,
             idx0, idx1, fb0, fb1, cb0, cb1,
             hia0, hia1, hib0, hib1,
             pld0, pld1, psc0, psc1, hld0, hld1, hsc0, hsc1,
             sh_f, sh_px, sh_py, sh_pz, sh_one, sh_cnt, sh_deg):
    c = lax.axis_index("c")
    s = lax.axis_index("s")
    lo = c * H2
    vecs = (sh_px, sh_py, sh_pz, sh_one, sh_cnt, sh_deg)

    _zero_rows(fb0, CHP, 128)
    pltpu.sync_copy(fb0, sh_f.at[pl.ds(s * 112, 80)])
    pltpu.sync_copy(fb0.at[pl.ds(0, 32)], sh_f.at[pl.ds(s * 112 + 80, 32)])
    _zero_vec(zb, 112)
    for t in vecs:
        pltpu.sync_copy(zb, t.at[pl.ds(s * 112, 112)])

    @pl.when(s == 0)
    def _():
        pltpu.sync_copy(fb0.at[pl.ds(0, 16)], sh_f.at[pl.ds(H2, 16)])
        for t in vecs:
            pltpu.sync_copy(zb.at[pl.ds(0, 16)], t.at[pl.ds(H2, 16)])

    plsc.subcore_barrier()

    _fill_ones(ones_buf, CHE)
    pslots = ((idx0, fb0, cb0, pld0, psc0), (idx1, fb1, cb1, pld1, psc1))

    def pdrain(fb, cb, sc):
        pltpu.make_async_copy(f12p.at[pl.ds(0, CHP)], fb, sc).wait()
        for q in range(4):
            pltpu.make_async_copy(px1.at[pl.ds(0, CHP)], cb.at[q], sc).wait()

    def pbody(k2, _):
        descs = {}
        for p, (idx, fb, cb, ld, sc) in enumerate(pslots):
            kk = 2 * k2 + p
            off = s * 800 + kk * CHP

            @pl.when(k2 > 0)
            def _():
                pdrain(fb, cb, sc)

            dl = [pltpu.async_copy(pids2.at[pl.ds(off, CHP)], idx, ld),
                  pltpu.async_copy(f12p.at[pl.ds(off, CHP)], fb, ld)]
            for q, colsrc in enumerate((px1, py1, pz1)):
                dl.append(pltpu.async_copy(colsrc.at[pl.ds(off, CHP)],
                                           cb.at[q], ld))
            descs[p] = dl
        for p, (idx, fb, cb, ld, sc) in enumerate(pslots):
            for dd in descs[p]:
                dd.wait()
            _localize(idx, CHP, lo, H2)
            pltpu.async_copy(fb, sh_f.at[idx], sc, add=True)
            for q, t in enumerate((sh_px, sh_py, sh_pz)):
                pltpu.async_copy(cb.at[q], t.at[idx], sc, add=True)
            pltpu.async_copy(ones_buf.at[pl.ds(0, CHP)], sh_one.at[idx],
                             sc, add=True)
        return _

    lax.fori_loop(0, 5, pbody, None)
    for idx, fb, cb, ld, sc in pslots:
        pdrain(fb, cb, sc)

    nk2 = (E2P // CHE // 16) // 2
    hslots = ((hia0, hib0, hld0, hsc0), (hia1, hib1, hld1, hsc1))

    def hdrain(sc):
        for _q in range(2):
            pltpu.make_async_copy(px1.at[pl.ds(0, CHE)], ones_buf, sc).wait()

    def hbody(k2, _):
        descs = {}
        for p, (hia, hib, ld, sc) in enumerate(hslots):
            kk = 2 * k2 + p
            off = (s + 16 * kk) * CHE

            @pl.when(k2 > 0)
            def _():
                hdrain(sc)

            descs[p] = [pltpu.async_copy(src.at[pl.ds(off, CHE)], hia, ld),
                        pltpu.async_copy(dst.at[pl.ds(off, CHE)], hib, ld)]
        for p, (hia, hib, ld, sc) in enumerate(hslots):
            for dd in descs[p]:
                dd.wait()
            _localize(hia, CHE, lo, H2)
            _localize(hib, CHE, lo, H2)
            pltpu.async_copy(ones_buf, sh_cnt.at[hia], sc, add=True)
            pltpu.async_copy(ones_buf, sh_deg.at[hib], sc, add=True)
        return _

    lax.fori_loop(0, nk2, hbody, None)
    for hia, hib, ld, sc in hslots:
        hdrain(sc)

    plsc.subcore_barrier()

    base = s * 112
    glob = c * H2 + s * 112
    ci = lax.broadcasted_iota(_I32, (16,), 0)

    pltpu.sync_copy(sh_f.at[pl.ds(base, 80)], fb0)
    pltpu.sync_copy(fb0, fsum2.at[pl.ds(glob, 80)])
    pltpu.sync_copy(sh_f.at[pl.ds(base + 80, 32)], fb0.at[pl.ds(0, 32)])
    pltpu.sync_copy(fb0.at[pl.ds(0, 32)], fsum2.at[pl.ds(glob + 80, 32)])

    pltpu.sync_copy(sh_px.at[pl.ds(base, 112)], pxb)
    pltpu.sync_copy(sh_py.at[pl.ds(base, 112)], pyb)
    pltpu.sync_copy(sh_pz.at[pl.ds(base, 112)], pzb)
    pltpu.sync_copy(sh_one.at[pl.ds(base, 112)], zb)
    pltpu.sync_copy(sh_cnt.at[pl.ds(base, 112)], cntb)
    pltpu.sync_copy(sh_deg.at[pl.ds(base, 112)], degb)
    pltpu.sync_copy(zb, pone.at[pl.ds(glob, 112)])

    def trow16(q, _):
        x16 = pxb[pl.ds(q * 16, 16)]
        y16 = pyb[pl.ds(q * 16, 16)]
        z16 = pzb[pl.ds(q * 16, 16)]
        o16 = zb[pl.ds(q * 16, 16)]
        c16 = cntb[pl.ds(q * 16, 16)]
        d16 = degb[pl.ds(q * 16, 16)]
        inv16 = 1.0 / jnp.maximum(o16, 1.0)
        xx = x16 * inv16
        yy = y16 * inv16
        zz = z16 * inv16
        zv = _zero16()
        for r16 in range(16):
            row = jnp.where(ci == 0, jnp.broadcast_to(xx[r16], (16,)), zv)
            row = jnp.where(ci == 1, jnp.broadcast_to(yy[r16], (16,)), row)
            row = jnp.where(ci == 2, jnp.broadcast_to(zz[r16], (16,)), row)
            row = jnp.where(ci == 7, jnp.broadcast_to(c16[r16], (16,)), row)
            row = jnp.where(ci == 8, jnp.broadcast_to(d16[r16], (16,)), row)
            tbuf[q * 16 + r16, pl.ds(0, 16)] = row
        return _

    lax.fori_loop(0, 7, trow16, None)
    pltpu.sync_copy(tbuf, table2.at[pl.ds(glob, 112)])


def _a2(f12p, pids2, px1, py1, pz1, src, dst):
    f = pl.kernel(
        _a2_body,
        out_type=[
            jax.ShapeDtypeStruct((M2P, 128), _F32),
            jax.ShapeDtypeStruct((M2P,), _F32),
            jax.ShapeDtypeStruct((M2P, 16), _F32),
        ],
        mesh=_mesh(),
        compiler_params=pltpu.CompilerParams(use_tc_tiling_on_sc=False),
        scratch_types=[
            pltpu.VMEM((CHE,), _F32),
            pltpu.VMEM((112,), _F32),
            pltpu.VMEM((112,), _F32),
            pltpu.VMEM((112,), _F32),
            pltpu.VMEM((112,), _F32),
            pltpu.VMEM((112,), _F32),
            pltpu.VMEM((112,), _F32),
            pltpu.VMEM((112, 16), _F32),
            pltpu.VMEM((CHP,), _I32),
            pltpu.VMEM((CHP,), _I32),
            pltpu.VMEM((CHP, 128), _F32),
            pltpu.VMEM((CHP, 128), _F32),
            pltpu.VMEM((4, CHP), _F32),
            pltpu.VMEM((4, CHP), _F32),
            pltpu.VMEM((CHE,), _I32),
            pltpu.VMEM((CHE,), _I32),
            pltpu.VMEM((CHE,), _I32),
            pltpu.VMEM((CHE,), _I32),
            pltpu.SemaphoreType.DMA,
            pltpu.SemaphoreType.DMA,
            pltpu.SemaphoreType.DMA,
            pltpu.SemaphoreType.DMA,
            pltpu.SemaphoreType.DMA,
            pltpu.SemaphoreType.DMA,
            pltpu.SemaphoreType.DMA,
            pltpu.SemaphoreType.DMA,
            pltpu.VMEM_SHARED((H2 + 16, 128), _F32),
            pltpu.VMEM_SHARED((H2 + 16,), _F32),
            pltpu.VMEM_SHARED((H2 + 16,), _F32),
            pltpu.VMEM_SHARED((H2 + 16,), _F32),
            pltpu.VMEM_SHARED((H2 + 16,), _F32),
            pltpu.VMEM_SHARED((H2 + 16,), _F32),
            pltpu.VMEM_SHARED((H2 + 16,), _F32),
        ],
    )
    return f(f12p, pids2, px1, py1, pz1, src, dst)


def _make_g_body(e):
    nk2 = (e // CHE // 32) // 2

    def body(tab, src, dst, gs, gd,
             sidx0, sidx1, didx0, didx1, rbs0, rbs1, rbd0, rbd1,
             ld0, ld1, g0, g1, wr0, wr1):
        c = lax.axis_index("c")
        s = lax.axis_index("s")
        wid = s * 2 + c
        slots = ((sidx0, didx0, rbs0, rbd0, ld0, g0, wr0),
                 (sidx1, didx1, rbs1, rbd1, ld1, g1, wr1))

        def wdrain(rbs, rbd, wr):
            pltpu.make_async_copy(tab.at[pl.ds(0, CHE)], rbs, wr).wait()
            pltpu.make_async_copy(tab.at[pl.ds(0, CHE)], rbd, wr).wait()

        def body2(k2, _):
            ldd = {}
            for p, (sidx, didx, rbs, rbd, ld, g, wr) in enumerate(slots):
                kk = 2 * k2 + p
                off = (wid + 32 * kk) * CHE

                @pl.when(k2 > 0)
                def _():
                    wdrain(rbs, rbd, wr)

                ldd[p] = [pltpu.async_copy(src.at[pl.ds(off, CHE)], sidx, ld),
                          pltpu.async_copy(dst.at[pl.ds(off, CHE)], didx, ld)]
            gdd = {}
            for p, (sidx, didx, rbs, rbd, ld, g, wr) in enumerate(slots):
                for dd in ldd[p]:
                    dd.wait()
                gdd[p] = [pltpu.async_copy(tab.at[sidx], rbs, g),
                          pltpu.async_copy(tab.at[didx], rbd, g)]
            for p, (sidx, didx, rbs, rbd, ld, g, wr) in enumerate(slots):
                kk = 2 * k2 + p
                off = (wid + 32 * kk) * CHE
                for dd in gdd[p]:
                    dd.wait()
                pltpu.async_copy(rbs, gs.at[pl.ds(off, CHE)], wr)
                pltpu.async_copy(rbd, gd.at[pl.ds(off, CHE)], wr)
            return _

        lax.fori_loop(0, nk2, body2, None)
        for sidx, didx, rbs, rbd, ld, g, wr in slots:
            wdrain(rbs, rbd, wr)

    return body


def _g(tab, src, dst, e):
    f = pl.kernel(
        _make_g_body(e),
        out_type=[
            jax.ShapeDtypeStruct((e, 16), _F32),
            jax.ShapeDtypeStruct((e, 16), _F32),
        ],
        mesh=_mesh(),
        compiler_params=pltpu.CompilerParams(use_tc_tiling_on_sc=False),
        scratch_types=[
            pltpu.VMEM((CHE,), _I32),
            pltpu.VMEM((CHE,), _I32),
            pltpu.VMEM((CHE,), _I32),
            pltpu.VMEM((CHE,), _I32),
            pltpu.VMEM((CHE, 16), _F32),
            pltpu.VMEM((CHE, 16), _F32),
            pltpu.VMEM((CHE, 16), _F32),
            pltpu.VMEM((CHE, 16), _F32),
            pltpu.SemaphoreType.DMA,
            pltpu.SemaphoreType.DMA,
            pltpu.SemaphoreType.DMA,
            pltpu.SemaphoreType.DMA,
            pltpu.SemaphoreType.DMA,
            pltpu.SemaphoreType.DMA,
        ],
    )
    return f(tab, src, dst)


def _make_c_body(mp, d, e):
    h = mp // 2
    rpt = h // 16
    nb = d // 16
    nk2 = (e // CHE // 16) // 2
    chunks = []
    o = 0
    while o < rpt:
        n = min(CHE, rpt - o)
        chunks.append((o, n))
        o += n

    def body(f_hbm, src, dst, s_e,
             agg,
             rows0, rows1, srcb0, srcb1, dstb0, dstb1, sb0, sb1,
             ld0, ld1, g0, g1, sc0, sc1,
             sh_agg):
        c = lax.axis_index("c")
        s = lax.axis_index("s")
        lo = c * h
        slots = ((srcb0, dstb0, sb0, rows0, ld0, g0, sc0),
                 (srcb1, dstb1, sb1, rows1, ld1, g1, sc1))

        _zero_rows(rows0, CHE, d)
        for o_, n_ in chunks:
            pltpu.sync_copy(rows0.at[pl.ds(0, n_)],
                            sh_agg.at[pl.ds(s * rpt + o_, n_)])

        @pl.when(s == 0)
        def _():
            pltpu.sync_copy(rows0.at[pl.ds(0, 16)], sh_agg.at[pl.ds(h, 16)])

        plsc.subcore_barrier()

        def sdrain(rows, sc):
            pltpu.make_async_copy(f_hbm.at[pl.ds(0, CHE)], rows, sc).wait()

        def body2(k2, _):
            ldd = {}
            for p, (srcb, dstb, sb, rows, ld, g, sc) in enumerate(slots):
                kk = 2 * k2 + p
                off = (s + 16 * kk) * CHE

                @pl.when(k2 > 0)
                def _():
                    sdrain(rows, sc)

                ldd[p] = [
                    pltpu.async_copy(src.at[pl.ds(off, CHE)], srcb, ld),
                    pltpu.async_copy(dst.at[pl.ds(off, CHE)], dstb, ld),
                    pltpu.async_copy(s_e.at[pl.ds(off, CHE)], sb, ld),
                ]
            gdd = {}
            for p, (srcb, dstb, sb, rows, ld, g, sc) in enumerate(slots):
                for dd in ldd[p]:
                    dd.wait()
                _localize(dstb, CHE, lo, h)
                gdd[p] = pltpu.async_copy(f_hbm.at[srcb], rows, g)
            for p, (srcb, dstb, sb, rows, ld, g, sc) in enumerate(slots):
                gdd[p].wait()

                def scale(r, _):
                    sv16 = sb[r, pl.ds(0, 16)]
                    for b_ in range(nb):
                        sl = pl.ds(b_ * 16, 16)
                        rows[r, sl] = rows[r, sl] * sv16
                    return _

                lax.fori_loop(0, CHE, scale, None)
                pltpu.async_copy(rows, sh_agg.at[dstb], sc, add=True)
            return _

        lax.fori_loop(0, nk2, body2, None)
        for srcb, dstb, sb, rows, ld, g, sc in slots:
            sdrain(rows, sc)
        plsc.subcore_barrier()

        for o_, n_ in chunks:
            pltpu.sync_copy(sh_agg.at[pl.ds(s * rpt + o_, n_)],
                            rows0.at[pl.ds(0, n_)])
            pltpu.sync_copy(rows0.at[pl.ds(0, n_)],
                            agg.at[pl.ds(lo + s * rpt + o_, n_)])

    return body


def _c(f_hbm, src, dst, s_e, mp, d, e):
    f = pl.kernel(
        _make_c_body(mp, d, e),
        out_type=jax.ShapeDtypeStruct((mp, d), _F32),
        mesh=_mesh(),
        compiler_params=pltpu.CompilerParams(use_tc_tiling_on_sc=False),
        scratch_types=[
            pltpu.VMEM((CHE, d), _F32),
            pltpu.VMEM((CHE, d), _F32),
            pltpu.VMEM((CHE,), _I32),
            pltpu.VMEM((CHE,), _I32),
            pltpu.VMEM((CHE,), _I32),
            pltpu.VMEM((CHE,), _I32),
            pltpu.VMEM((CHE, 16), _F32),
            pltpu.VMEM((CHE, 16), _F32),
            pltpu.SemaphoreType.DMA,
            pltpu.SemaphoreType.DMA,
            pltpu.SemaphoreType.DMA,
            pltpu.SemaphoreType.DMA,
            pltpu.SemaphoreType.DMA,
            pltpu.SemaphoreType.DMA,
            pltpu.VMEM_SHARED((mp // 2 + 16, d), _F32),
        ],
    )
    return f(f_hbm, src, dst, s_e)


def _bn_relu(x, g, b):
    m = jnp.mean(x, axis=0)
    v = jnp.mean((x - m) ** 2, axis=0)
    return jnp.maximum((x - m) / jnp.sqrt(v + 1e-5) * g + b, 0.0)


def _make_tcb_body(inv_radius):
    def body(gs_ref, gd_ref, kaa_ref, kba_ref, kab_ref, kbb_ref,
             sa_ref, sb_ref):
        gs = gs_ref[...]
        gd = gd_ref[...]
        rel = (gd[:, 0:3] - gs[:, 0:3]) * inv_radius
        wa = jnp.dot(jnp.maximum(jnp.dot(rel, kaa_ref[...],
                                         preferred_element_type=_F32), 0.0),
                     kba_ref[...], preferred_element_type=_F32)
        wb = jnp.dot(jnp.maximum(jnp.dot(rel, kab_ref[...],
                                         preferred_element_type=_F32), 0.0),
                     kbb_ref[...], preferred_element_type=_F32)
        invp = 1.0 / jnp.maximum(gs[:, 7:8] * (1.0 / 32.0), 1e-3)
        invd = 1.0 / jnp.maximum(gd[:, 8:9], 1.0)
        sc = invp * invd
        blk = gs.shape[0]
        sa_ref[...] = jnp.broadcast_to(wa * sc, (blk, 16))
        sb_ref[...] = jnp.broadcast_to(wb * sc, (blk, 16))

    return body


_TCB_BLK = 8192


def _tcb(gs, gd, kaa, kba, kab, kbb, e, inv_radius):
    nblk = e // _TCB_BLK
    out = pl.pallas_call(
        _make_tcb_body(inv_radius),
        grid=(nblk,),
        in_specs=[
            pl.BlockSpec((_TCB_BLK, 16), lambda i: (i, 0)),
            pl.BlockSpec((_TCB_BLK, 16), lambda i: (i, 0)),
            pl.BlockSpec((3, 16), lambda i: (0, 0)),
            pl.BlockSpec((16, 1), lambda i: (0, 0)),
            pl.BlockSpec((3, 16), lambda i: (0, 0)),
            pl.BlockSpec((16, 1), lambda i: (0, 0)),
        ],
        out_specs=[
            pl.BlockSpec((_TCB_BLK, 16), lambda i: (i, 0)),
            pl.BlockSpec((_TCB_BLK, 16), lambda i: (i, 0)),
        ],
        out_shape=[
            jax.ShapeDtypeStruct((e, 16), _F32),
            jax.ShapeDtypeStruct((e, 16), _F32),
        ],
    )(gs, gd, kaa, kba, kab, kbb)
    return out[0], out[1]


def _tc1_body(agg_ref, w_ref, g_ref, b_ref, out_ref):
    a = agg_ref[...]
    x = jnp.dot(a[:M1, 3:6], w_ref[...], preferred_element_type=_F32)
    out_ref[...] = _bn_relu(x, g_ref[...], b_ref[...])


def _tc2_body(agg_ref, w_ref, out_ref):
    a = agg_ref[...]
    y = jnp.dot(a[:M1], w_ref[...], preferred_element_type=_F32)
    out_ref[...] = jnp.concatenate(
        [y, jnp.zeros((M1P - M1, 128), _F32)], axis=0)


def _tc3_body(fsum_ref, cnt_ref, g_ref, b_ref, out_ref):
    f2 = fsum_ref[:M2] / jnp.maximum(cnt_ref[0, :M2], 1.0)[:, None]
    out_ref[...] = _bn_relu(f2, g_ref[...], b_ref[...])


def _tc4_body(agg_ref, w_ref, g_ref, b_ref, out_ref):
    a = agg_ref[...]
    x = jnp.dot(a[:M2], w_ref[...], preferred_element_type=_F32)
    out_ref[...] = _bn_relu(x, g_ref[...], b_ref[...])


def _tc5_body(agg_ref, w_ref, ids_ref, g4_ref, b4_ref, w1_ref, c1_ref,
              g5_ref, b5_ref, w2_ref, c2_ref, out_ref):
    a = agg_ref[...]
    f4 = jnp.dot(a[:M2], w_ref[...], preferred_element_type=_F32)
    seg = lax.broadcasted_iota(_I32, (NB, M2), 0)
    onehot = (ids_ref[...] == seg).astype(_F32)
    cnt = jnp.sum(onehot, axis=1, keepdims=True)
    gm = jnp.dot(onehot, f4, preferred_element_type=_F32)
    gm = gm / jnp.maximum(cnt, 1.0)
    gm = _bn_relu(gm, g4_ref[...], b4_ref[...])
    gm = jnp.dot(gm, w1_ref[...], preferred_element_type=_F32) + c1_ref[...]
    gm = _bn_relu(gm, g5_ref[...], b5_ref[...])
    out_ref[...] = (jnp.dot(gm, w2_ref[...], preferred_element_type=_F32)
                    + c2_ref[...])


def _tc(body, out_shape, *args):
    return pl.pallas_call(
        body, out_shape=jax.ShapeDtypeStruct(out_shape, _F32))(*args)


def kernel(pts, feats, pool_ids1, edge_index1, pool_ids2, edge_index2,
           batch_ids_out,
           k1a_11, k1b_11, w_11, bn_g1, bn_b1, k1a_12, k1b_12, w_12, bn_g2,
           bn_b2, k1a_21, k1b_21, w_21, bn_g3, bn_b3, k1a_22, k1b_22, w_22,
           bn_g4, bn_b4, fc1_w, fc1_b, bn_g5, bn_b5, fc2_w, fc2_b):
    ones_col = jnp.ones((N, 1), _F32)
    rows16 = jnp.concatenate(
        [pts, feats, ones_col, jnp.zeros((N, 9), _F32)], axis=1)
    rows16 = jnp.concatenate(
        [rows16, jnp.zeros((NP - N, 16), _F32)], axis=0)
    ids1 = jnp.concatenate(
        [pool_ids1.astype(_I32), jnp.zeros((NP - N,), _I32)])
    pe1 = jnp.full((E1P - E1,), M1, _I32)
    src1 = jnp.concatenate([edge_index1[0].astype(_I32), pe1])
    dst1 = jnp.concatenate([edge_index1[1].astype(_I32), pe1])
    pe2 = jnp.full((E2P - E2,), M2, _I32)
    src2 = jnp.concatenate([edge_index2[0].astype(_I32), pe2])
    dst2 = jnp.concatenate([edge_index2[1].astype(_I32), pe2])
    pids2 = jnp.concatenate(
        [pool_ids2.astype(_I32),
         jnp.full((M1P - M1,), M2P - 1, _I32)])
    pad0 = jnp.zeros((NP - N,), _F32)
    pxc = jnp.concatenate([pts[:, 0], pad0])
    pyc = jnp.concatenate([pts[:, 1], pad0])
    pzc = jnp.concatenate([pts[:, 2], pad0])
    onec = jnp.concatenate([jnp.ones((N,), _F32), pad0])

    table16, px1, py1, pz1 = _a1(
        rows16, ids1, src1, dst1, pxc, pyc, pzc, onec)
    gs1, gd1 = _g(table16, src1, dst1, e=E1P)
    s11, s12 = _tcb(gs1, gd1, k1a_11, k1b_11, k1a_12, k1b_12,
                    e=E1P, inv_radius=10.0)
    agg11 = _c(table16, src1, dst1, s11, mp=M1P, d=16, e=E1P)
    f1 = _tc(_tc1_body, (M1, 128), agg11, w_11, bn_g1, bn_b1)
    agg12 = _c(f1, src1, dst1, s12, mp=M1P, d=128, e=E1P)
    f12p = _tc(_tc2_body, (M1P, 128), agg12, w_12)

    fsum2, pone, table2 = _a2(f12p, pids2, px1, py1, pz1, src2, dst2)
    f2 = _tc(_tc3_body, (M2, 128), fsum2, pone.reshape(1, M2P),
             bn_g2, bn_b2)

    gs2, gd2 = _g(table2, src2, dst2, e=E2P)
    s21, s22 = _tcb(gs2, gd2, k1a_21, k1b_21, k1a_22, k1b_22,
                    e=E2P, inv_radius=5.0)
    agg21 = _c(f2, src2, dst2, s21, mp=M2P, d=128, e=E2P)
    f3 = _tc(_tc4_body, (M2, 128), agg21, w_21, bn_g3, bn_b3)
    agg22 = _c(f3, src2, dst2, s22, mp=M2P, d=128, e=E2P)

    ids_out = batch_ids_out.astype(_I32).reshape(1, M2)
    return _tc(_tc5_body, (NB, 40), agg22, w_22, ids_out, bn_g4, bn_b4,
               fc1_w, fc1_b, bn_g5, bn_b5, fc2_w, fc2_b)


# speedup vs baseline: 3.2303x; 1.0115x over previous
import jax
import jax.numpy as jnp
from jax import lax
from jax.experimental import pallas as pl
from jax.experimental.pallas import tpu as pltpu
from jax.experimental.pallas import tpu_sc as plsc

N = 50000
NP = 51200
M1 = 12500
M1P = 12800
H1 = 6400
M2 = 3125
M2P = 3584
H2 = 1792
E1 = 400000
E1P = 401408
E2 = 100000
E2P = 106496
NB = 8
CHE = 128
CHP = 80

_F32 = jnp.float32
_I32 = jnp.int32


def _mesh():
    return plsc.VectorSubcoreMesh(core_axis_name="c", subcore_axis_name="s")


def _zero16():
    return jnp.zeros((16,), _F32)


def _zero_rows(ref, nrows, ncols):
    nb = ncols // 16

    def body(i, _):
        for b in range(nb):
            ref[i, pl.ds(b * 16, 16)] = _zero16()
        return _

    lax.fori_loop(0, nrows, body, None)


def _zero_vec(ref, n):
    def body(i, _):
        ref[pl.ds(i * 16, 16)] = _zero16()
        return _

    lax.fori_loop(0, n // 16, body, None)


def _fill_ones(ref, n):
    one = jnp.ones((16,), _F32)

    def body(i, _):
        ref[pl.ds(i * 16, 16)] = one
        return _

    lax.fori_loop(0, n // 16, body, None)


def _localize(idx_ref, n, lo, h):
    def body(q, _):
        sl = pl.ds(q * 16, 16)
        v = idx_ref[sl] - lo
        ok = (v >= 0) & (v < h)
        idx_ref[sl] = jnp.where(ok, v, h)
        return _

    lax.fori_loop(0, n // 16, body, None)


def _a1_body(rows16, ids, src, dst, pxc, pyc, pzc, onec,
             table16, px1, py1, pz1,
             slab, ones_buf, zb, pxb, pyb, pzb, cntb, degb,
             idx0, idx1, rowb0, rowb1, cb0, cb1,
             hia0, hia1, hib0, hib1,
             pld0, pld1, psc0, psc1, hld0, hld1, hsc0, hsc1,
             sh_table, sh_cnt, sh_deg, sh_px, sh_py, sh_pz, sh_one):
    c = lax.axis_index("c")
    s = lax.axis_index("s")
    lo = c * H1
    vecs = (sh_cnt, sh_deg, sh_px, sh_py, sh_pz, sh_one)

    _zero_rows(slab, 400, 16)
    pltpu.sync_copy(slab, sh_table.at[pl.ds(s * 400, 400)])
    _zero_vec(zb, 400)
    for t in vecs:
        pltpu.sync_copy(zb, t.at[pl.ds(s * 400, 400)])

    @pl.when(s == 0)
    def _():
        pltpu.sync_copy(slab.at[pl.ds(0, 16)], sh_table.at[pl.ds(H1, 16)])
        for t in vecs:
            pltpu.sync_copy(zb.at[pl.ds(0, 16)], t.at[pl.ds(H1, 16)])

    plsc.subcore_barrier()

    pslots = ((idx0, rowb0, cb0, pld0, psc0), (idx1, rowb1, cb1, pld1, psc1))

    def pdrain(rowb, cb, sc):
        pltpu.make_async_copy(rows16.at[pl.ds(0, CHP)], rowb, sc).wait()
        for q in range(4):
            pltpu.make_async_copy(pxc.at[pl.ds(0, CHP)], cb.at[q], sc).wait()

    def pbody(k2, _):
        descs = {}
        for p, (idx, rowb, cb, ld, sc) in enumerate(pslots):
            kk = 2 * k2 + p
            off = s * 3200 + kk * CHP

            @pl.when(k2 > 0)
            def _():
                pdrain(rowb, cb, sc)

            dl = [pltpu.async_copy(ids.at[pl.ds(off, CHP)], idx, ld),
                  pltpu.async_copy(rows16.at[pl.ds(off, CHP)], rowb, ld)]
            for q, colsrc in enumerate((pxc, pyc, pzc, onec)):
                dl.append(pltpu.async_copy(colsrc.at[pl.ds(off, CHP)],
                                           cb.at[q], ld))
            descs[p] = dl
        for p, (idx, rowb, cb, ld, sc) in enumerate(pslots):
            for dd in descs[p]:
                dd.wait()
            _localize(idx, CHP, lo, H1)
            pltpu.async_copy(rowb, sh_table.at[idx], sc, add=True)
            for q, t in enumerate((sh_px, sh_py, sh_pz, sh_one)):
                pltpu.async_copy(cb.at[q], t.at[idx], sc, add=True)
        return _

    lax.fori_loop(0, 20, pbody, None)
    for idx, rowb, cb, ld, sc in pslots:
        pdrain(rowb, cb, sc)

    _fill_ones(ones_buf, CHE)
    nk2 = (E1P // CHE // 16) // 2
    hslots = ((hia0, hib0, hld0, hsc0), (hia1, hib1, hld1, hsc1))

    def hdrain(sc):
        for _q in range(2):
            pltpu.make_async_copy(onec.at[pl.ds(0, CHE)], ones_buf, sc).wait()

    def hbody(k2, _):
        descs = {}
        for p, (hia, hib, ld, sc) in enumerate(hslots):
            kk = 2 * k2 + p
            off = (s + 16 * kk) * CHE

            @pl.when(k2 > 0)
            def _():
                hdrain(sc)

            descs[p] = [pltpu.async_copy(src.at[pl.ds(off, CHE)], hia, ld),
                        pltpu.async_copy(dst.at[pl.ds(off, CHE)], hib, ld)]
        for p, (hia, hib, ld, sc) in enumerate(hslots):
            for dd in descs[p]:
                dd.wait()
            _localize(hia, CHE, lo, H1)
            _localize(hib, CHE, lo, H1)
            pltpu.async_copy(ones_buf, sh_cnt.at[hia], sc, add=True)
            pltpu.async_copy(ones_buf, sh_deg.at[hib], sc, add=True)
        return _

    lax.fori_loop(0, nk2, hbody, None)
    for hia, hib, ld, sc in hslots:
        hdrain(sc)

    plsc.subcore_barrier()

    base = s * 400
    glob = c * H1 + s * 400
    ci = lax.broadcasted_iota(_I32, (16,), 0)

    pltpu.sync_copy(sh_table.at[pl.ds(base, 400)], slab)
    pltpu.sync_copy(sh_cnt.at[pl.ds(base, 400)], cntb)
    pltpu.sync_copy(sh_deg.at[pl.ds(base, 400)], degb)

    def drow16(q, _):
        cnt16 = cntb[pl.ds(q * 16, 16)]
        deg16 = degb[pl.ds(q * 16, 16)]
        for r16 in range(16):
            r = q * 16 + r16
            rowv = slab[r, pl.ds(0, 16)]
            invv = 1.0 / jnp.maximum(jnp.broadcast_to(rowv[6], (16,)), 1.0)
            out = rowv * invv
            out = jnp.where(ci == 7, jnp.broadcast_to(cnt16[r16], (16,)), out)
            out = jnp.where(ci == 8, jnp.broadcast_to(deg16[r16], (16,)), out)
            slab[r, pl.ds(0, 16)] = out
        return _

    lax.fori_loop(0, 25, drow16, None)
    pltpu.sync_copy(slab, table16.at[pl.ds(glob, 400)])

    pltpu.sync_copy(sh_px.at[pl.ds(base, 400)], pxb)
    pltpu.sync_copy(sh_py.at[pl.ds(base, 400)], pyb)
    pltpu.sync_copy(sh_pz.at[pl.ds(base, 400)], pzb)
    pltpu.sync_copy(sh_one.at[pl.ds(base, 400)], zb)

    def dv(q, _):
        sl = pl.ds(q * 16, 16)
        inv = 1.0 / jnp.maximum(zb[sl], 1.0)
        pxb[sl] = pxb[sl] * inv
        pyb[sl] = pyb[sl] * inv
        pzb[sl] = pzb[sl] * inv
        return _

    lax.fori_loop(0, 25, dv, None)

    pltpu.sync_copy(pxb, px1.at[pl.ds(glob, 400)])
    pltpu.sync_copy(pyb, py1.at[pl.ds(glob, 400)])
    pltpu.sync_copy(pzb, pz1.at[pl.ds(glob, 400)])


def _a1(rows16, ids, src, dst, pxc, pyc, pzc, onec):
    f = pl.kernel(
        _a1_body,
        out_type=[
            jax.ShapeDtypeStruct((M1P, 16), _F32),
            jax.ShapeDtypeStruct((M1P,), _F32),
            jax.ShapeDtypeStruct((M1P,), _F32),
            jax.ShapeDtypeStruct((M1P,), _F32),
        ],
        mesh=_mesh(),
        compiler_params=pltpu.CompilerParams(use_tc_tiling_on_sc=False),
        scratch_types=[
            pltpu.VMEM((400, 16), _F32),
            pltpu.VMEM((CHE,), _F32),
            pltpu.VMEM((400,), _F32),
            pltpu.VMEM((400,), _F32),
            pltpu.VMEM((400,), _F32),
            pltpu.VMEM((400,), _F32),
            pltpu.VMEM((400,), _F32),
            pltpu.VMEM((400,), _F32),
            pltpu.VMEM((CHP,), _I32),
            pltpu.VMEM((CHP,), _I32),
            pltpu.VMEM((CHP, 16), _F32),
            pltpu.VMEM((CHP, 16), _F32),
            pltpu.VMEM((4, CHP), _F32),
            pltpu.VMEM((4, CHP), _F32),
            pltpu.VMEM((CHE,), _I32),
            pltpu.VMEM((CHE,), _I32),
            pltpu.VMEM((CHE,), _I32),
            pltpu.VMEM((CHE,), _I32),
            pltpu.SemaphoreType.DMA,
            pltpu.SemaphoreType.DMA,
            pltpu.SemaphoreType.DMA,
            pltpu.SemaphoreType.DMA,
            pltpu.SemaphoreType.DMA,
            pltpu.SemaphoreType.DMA,
            pltpu.SemaphoreType.DMA,
            pltpu.SemaphoreType.DMA,
            pltpu.VMEM_SHARED((H1 + 16, 16), _F32),
            pltpu.VMEM_SHARED((H1 + 16,), _F32),
            pltpu.VMEM_SHARED((H1 + 16,), _F32),
            pltpu.VMEM_SHARED((H1 + 16,), _F32),
            pltpu.VMEM_SHARED((H1 + 16,), _F32),
            pltpu.VMEM_SHARED((H1 + 16,), _F32),
            pltpu.VMEM_SHARED((H1 + 16,), _F32),
        ],
    )
    return f(rows16, ids, src, dst, pxc, pyc, pzc, onec)


def _a2_body(f12p, pids2, px1, py1, pz1, src, dst,
             fsum2, pone, table2,
             ones_buf, zb, pxb, pyb, pzb, cntb, degb, tbuf,
             idx0, idx1, fb0, fb1, cb0, cb1,
             hia0, hia1, hib0, hib1,
             pld0, pld1, psc0, psc1, hld0, hld1, hsc0, hsc1,
             sh_f, sh_px, sh_py, sh_pz, sh_one, sh_cnt, sh_deg):
    c = lax.axis_index("c")
    s = lax.axis_index("s")
    lo = c * H2
    vecs = (sh_px, sh_py, sh_pz, sh_one, sh_cnt, sh_deg)

    _zero_rows(fb0, CHP, 128)
    pltpu.sync_copy(fb0, sh_f.at[pl.ds(s * 112, 80)])
    pltpu.sync_copy(fb0.at[pl.ds(0, 32)], sh_f.at[pl.ds(s * 112 + 80, 32)])
    _zero_vec(zb, 112)
    for t in vecs:
        pltpu.sync_copy(zb, t.at[pl.ds(s * 112, 112)])

    @pl.when(s == 0)
    def _():
        pltpu.sync_copy(fb0.at[pl.ds(0, 16)], sh_f.at[pl.ds(H2, 16)])
        for t in vecs:
            pltpu.sync_copy(zb.at[pl.ds(0, 16)], t.at[pl.ds(H2, 16)])

    plsc.subcore_barrier()

    _fill_ones(ones_buf, CHE)
    pslots = ((idx0, fb0, cb0, pld0, psc0), (idx1, fb1, cb1, pld1, psc1))

    def pdrain(fb, cb, sc):
        pltpu.make_async_copy(f12p.at[pl.ds(0, CHP)], fb, sc).wait()
        for q in range(4):
            pltpu.make_async_copy(px1.at[pl.ds(0, CHP)], cb.at[q], sc).wait()

    def pbody(k2, _):
        descs = {}
        for p, (idx, fb, cb, ld, sc) in enumerate(pslots):
            kk = 2 * k2 + p
            off = s * 800 + kk * CHP

            @pl.when(k2 > 0)
            def _():
                pdrain(fb, cb, sc)

            dl = [pltpu.async_copy(pids2.at[pl.ds(off, CHP)], idx, ld),
                  pltpu.async_copy(f12p.at[pl.ds(off, CHP)], fb, ld)]
            for q, colsrc in enumerate((px1, py1, pz1)):
                dl.append(pltpu.async_copy(colsrc.at[pl.ds(off, CHP)],
                                           cb.at[q], ld))
            descs[p] = dl
        for p, (idx, fb, cb, ld, sc) in enumerate(pslots):
            for dd in descs[p]:
                dd.wait()
            _localize(idx, CHP, lo, H2)
            pltpu.async_copy(fb, sh_f.at[idx], sc, add=True)
            for q, t in enumerate((sh_px, sh_py, sh_pz)):
                pltpu.async_copy(cb.at[q], t.at[idx], sc, add=True)
            pltpu.async_copy(ones_buf.at[pl.ds(0, CHP)], sh_one.at[idx],
                             sc, add=True)
        return _

    lax.fori_loop(0, 5, pbody, None)
    for idx, fb, cb, ld, sc in pslots:
        pdrain(fb, cb, sc)

    nk2 = (E2P // CHE // 16) // 2
    hslots = ((hia0, hib0, hld0, hsc0), (hia1, hib1, hld1, hsc1))

    def hdrain(sc):
        for _q in range(2):
            pltpu.make_async_copy(px1.at[pl.ds(0, CHE)], ones_buf, sc).wait()

    def hbody(k2, _):
        descs = {}
        for p, (hia, hib, ld, sc) in enumerate(hslots):
            kk = 2 * k2 + p
            off = (s + 16 * kk) * CHE

            @pl.when(k2 > 0)
            def _():
                hdrain(sc)

            descs[p] = [pltpu.async_copy(src.at[pl.ds(off, CHE)], hia, ld),
                        pltpu.async_copy(dst.at[pl.ds(off, CHE)], hib, ld)]
        for p, (hia, hib, ld, sc) in enumerate(hslots):
            for dd in descs[p]:
                dd.wait()
            _localize(hia, CHE, lo, H2)
            _localize(hib, CHE, lo, H2)
            pltpu.async_copy(ones_buf, sh_cnt.at[hia], sc, add=True)
            pltpu.async_copy(ones_buf, sh_deg.at[hib], sc, add=True)
        return _

    lax.fori_loop(0, nk2, hbody, None)
    for hia, hib, ld, sc in hslots:
        hdrain(sc)

    plsc.subcore_barrier()

    base = s * 112
    glob = c * H2 + s * 112
    ci = lax.broadcasted_iota(_I32, (16,), 0)

    pltpu.sync_copy(sh_f.at[pl.ds(base, 80)], fb0)
    pltpu.sync_copy(fb0, fsum2.at[pl.ds(glob, 80)])
    pltpu.sync_copy(sh_f.at[pl.ds(base + 80, 32)], fb0.at[pl.ds(0, 32)])
    pltpu.sync_copy(fb0.at[pl.ds(0, 32)], fsum2.at[pl.ds(glob + 80, 32)])

    pltpu.sync_copy(sh_px.at[pl.ds(base, 112)], pxb)
    pltpu.sync_copy(sh_py.at[pl.ds(base, 112)], pyb)
    pltpu.sync_copy(sh_pz.at[pl.ds(base, 112)], pzb)
    pltpu.sync_copy(sh_one.at[pl.ds(base, 112)], zb)
    pltpu.sync_copy(sh_cnt.at[pl.ds(base, 112)], cntb)
    pltpu.sync_copy(sh_deg.at[pl.ds(base, 112)], degb)
    pltpu.sync_copy(zb, pone.at[pl.ds(glob, 112)])

    def trow16(q, _):
        x16 = pxb[pl.ds(q * 16, 16)]
        y16 = pyb[pl.ds(q * 16, 16)]
        z16 = pzb[pl.ds(q * 16, 16)]
        o16 = zb[pl.ds(q * 16, 16)]
        c16 = cntb[pl.ds(q * 16, 16)]
        d16 = degb[pl.ds(q * 16, 16)]
        inv16 = 1.0 / jnp.maximum(o16, 1.0)
        xx = x16 * inv16
        yy = y16 * inv16
        zz = z16 * inv16
        zv = _zero16()
        for r16 in range(16):
            row = jnp.where(ci == 0, jnp.broadcast_to(xx[r16], (16,)), zv)
            row = jnp.where(ci == 1, jnp.broadcast_to(yy[r16], (16,)), row)
            row = jnp.where(ci == 2, jnp.broadcast_to(zz[r16], (16,)), row)
            row = jnp.where(ci == 7, jnp.broadcast_to(c16[r16], (16,)), row)
            row = jnp.where(ci == 8, jnp.broadcast_to(d16[r16], (16,)), row)
            tbuf[q * 16 + r16, pl.ds(0, 16)] = row
        return _

    lax.fori_loop(0, 7, trow16, None)
    pltpu.sync_copy(tbuf, table2.at[pl.ds(glob, 112)])


def _a2(f12p, pids2, px1, py1, pz1, src, dst):
    f = pl.kernel(
        _a2_body,
        out_type=[
            jax.ShapeDtypeStruct((M2P, 128), _F32),
            jax.ShapeDtypeStruct((M2P,), _F32),
            jax.ShapeDtypeStruct((M2P, 16), _F32),
        ],
        mesh=_mesh(),
        compiler_params=pltpu.CompilerParams(use_tc_tiling_on_sc=False),
        scratch_types=[
            pltpu.VMEM((CHE,), _F32),
            pltpu.VMEM((112,), _F32),
            pltpu.VMEM((112,), _F32),
            pltpu.VMEM((112,), _F32),
            pltpu.VMEM((112,), _F32),
            pltpu.VMEM((112,), _F32),
            pltpu.VMEM((112,), _F32),
            pltpu.VMEM((112, 16), _F32),
            pltpu.VMEM((CHP,), _I32),
            pltpu.VMEM((CHP,), _I32),
            pltpu.VMEM((CHP, 128), _F32),
            pltpu.VMEM((CHP, 128), _F32),
            pltpu.VMEM((4, CHP), _F32),
            pltpu.VMEM((4, CHP), _F32),
            pltpu.VMEM((CHE,), _I32),
            pltpu.VMEM((CHE,), _I32),
            pltpu.VMEM((CHE,), _I32),
            pltpu.VMEM((CHE,), _I32),
            pltpu.SemaphoreType.DMA,
            pltpu.SemaphoreType.DMA,
            pltpu.SemaphoreType.DMA,
            pltpu.SemaphoreType.DMA,
            pltpu.SemaphoreType.DMA,
            pltpu.SemaphoreType.DMA,
            pltpu.SemaphoreType.DMA,
            pltpu.SemaphoreType.DMA,
            pltpu.VMEM_SHARED((H2 + 16, 128), _F32),
            pltpu.VMEM_SHARED((H2 + 16,), _F32),
            pltpu.VMEM_SHARED((H2 + 16,), _F32),
            pltpu.VMEM_SHARED((H2 + 16,), _F32),
            pltpu.VMEM_SHARED((H2 + 16,), _F32),
            pltpu.VMEM_SHARED((H2 + 16,), _F32),
            pltpu.VMEM_SHARED((H2 + 16,), _F32),
        ],
    )
    return f(f12p, pids2, px1, py1, pz1, src, dst)


def _make_g_body(e):
    nk2 = (e // CHE // 32) // 2

    def body(tab, src, dst, gs, gd,
             sidx0, sidx1, didx0, didx1, rbs0, rbs1, rbd0, rbd1,
             ld0, ld1, g0, g1, wr0, wr1):
        c = lax.axis_index("c")
        s = lax.axis_index("s")
        wid = s * 2 + c
        slots = ((sidx0, didx0, rbs0, rbd0, ld0, g0, wr0),
                 (sidx1, didx1, rbs1, rbd1, ld1, g1, wr1))

        def wdrain(rbs, rbd, wr):
            pltpu.make_async_copy(tab.at[pl.ds(0, CHE)], rbs, wr).wait()
            pltpu.make_async_copy(tab.at[pl.ds(0, CHE)], rbd, wr).wait()

        def body2(k2, _):
            ldd = {}
            for p, (sidx, didx, rbs, rbd, ld, g, wr) in enumerate(slots):
                kk = 2 * k2 + p
                off = (wid + 32 * kk) * CHE

                @pl.when(k2 > 0)
                def _():
                    wdrain(rbs, rbd, wr)

                ldd[p] = [pltpu.async_copy(src.at[pl.ds(off, CHE)], sidx, ld),
                          pltpu.async_copy(dst.at[pl.ds(off, CHE)], didx, ld)]
            gdd = {}
            for p, (sidx, didx, rbs, rbd, ld, g, wr) in enumerate(slots):
                for dd in ldd[p]:
                    dd.wait()
                gdd[p] = [pltpu.async_copy(tab.at[sidx], rbs, g),
                          pltpu.async_copy(tab.at[didx], rbd, g)]
            for p, (sidx, didx, rbs, rbd, ld, g, wr) in enumerate(slots):
                kk = 2 * k2 + p
                off = (wid + 32 * kk) * CHE
                for dd in gdd[p]:
                    dd.wait()
                pltpu.async_copy(rbs, gs.at[pl.ds(off, CHE)], wr)
                pltpu.async_copy(rbd, gd.at[pl.ds(off, CHE)], wr)
            return _

        lax.fori_loop(0, nk2, body2, None)
        for sidx, didx, rbs, rbd, ld, g, wr in slots:
            wdrain(rbs, rbd, wr)

    return body


def _g(tab, src, dst, e):
    f = pl.kernel(
        _make_g_body(e),
        out_type=[
            jax.ShapeDtypeStruct((e, 16), _F32),
            jax.ShapeDtypeStruct((e, 16), _F32),
        ],
        mesh=_mesh(),
        compiler_params=pltpu.CompilerParams(use_tc_tiling_on_sc=False),
        scratch_types=[
            pltpu.VMEM((CHE,), _I32),
            pltpu.VMEM((CHE,), _I32),
            pltpu.VMEM((CHE,), _I32),
            pltpu.VMEM((CHE,), _I32),
            pltpu.VMEM((CHE, 16), _F32),
            pltpu.VMEM((CHE, 16), _F32),
            pltpu.VMEM((CHE, 16), _F32),
            pltpu.VMEM((CHE, 16), _F32),
            pltpu.SemaphoreType.DMA,
            pltpu.SemaphoreType.DMA,
            pltpu.SemaphoreType.DMA,
            pltpu.SemaphoreType.DMA,
            pltpu.SemaphoreType.DMA,
            pltpu.SemaphoreType.DMA,
        ],
    )
    return f(tab, src, dst)


def _make_c_body(mp, d, e, nrep, premul):
    h = mp // 2
    rpt = h // 16
    nb = d // 16
    nk2 = (e // CHE // 16) // 2
    chunks = []
    o = 0
    while o < rpt:
        n = min(CHE, rpt - o)
        chunks.append((o, n))
        o += n

    def body(f_hbm, src, dst, s_e, agg, *rest):
        (rows0, rows1, srcb0, srcb1, dstb0, dstb1, sb0, sb1,
         ld0, ld1, g0, g1, sc0, sc1) = rest[:14]
        shs = rest[14:]
        c = lax.axis_index("c")
        s = lax.axis_index("s")
        lo = c * h
        slots = ((srcb0, dstb0, sb0, rows0, ld0, g0, sc0),
                 (srcb1, dstb1, sb1, rows1, ld1, g1, sc1))

        _zero_rows(rows0, CHE, d)
        for sh in shs:
            for o_, n_ in chunks:
                pltpu.sync_copy(rows0.at[pl.ds(0, n_)],
                                sh.at[pl.ds(s * rpt + o_, n_)])

            @pl.when(s == 0)
            def _():
                pltpu.sync_copy(rows0.at[pl.ds(0, 16)], sh.at[pl.ds(h, 16)])

        plsc.subcore_barrier()

        def sdrain(rows, sc):
            pltpu.make_async_copy(f_hbm.at[pl.ds(0, CHE)], rows, sc).wait()

        def scatter_add(rows, dstb, sc):
            if nrep == 1:
                pltpu.async_copy(rows, shs[0].at[dstb], sc, add=True)
            else:
                @pl.when(s % 2 == 0)
                def _():
                    pltpu.async_copy(rows, shs[0].at[dstb], sc, add=True)

                @pl.when(s % 2 == 1)
                def _():
                    pltpu.async_copy(rows, shs[1].at[dstb], sc, add=True)

        def body2(k2, _):
            ldd = {}
            for p, (srcb, dstb, sb, rows, ld, g, sc) in enumerate(slots):
                kk = 2 * k2 + p
                off = (s + 16 * kk) * CHE

                @pl.when(k2 > 0)
                def _():
                    sdrain(rows, sc)

                dl = [pltpu.async_copy(dst.at[pl.ds(off, CHE)], dstb, ld)]
                if premul:
                    dl.append(pltpu.async_copy(f_hbm.at[pl.ds(off, CHE)],
                                               rows, g))
                else:
                    dl.append(pltpu.async_copy(src.at[pl.ds(off, CHE)],
                                               srcb, ld))
                    dl.append(pltpu.async_copy(s_e.at[pl.ds(off, CHE)],
                                               sb, ld))
                ldd[p] = dl
            gdd = {}
            for p, (srcb, dstb, sb, rows, ld, g, sc) in enumerate(slots):
                for dd in ldd[p]:
                    dd.wait()
                _localize(dstb, CHE, lo, h)
                if not premul:
                    gdd[p] = pltpu.async_copy(f_hbm.at[srcb], rows, g)
            for p, (srcb, dstb, sb, rows, ld, g, sc) in enumerate(slots):
                if not premul:
                    gdd[p].wait()

                    def scale(r, _):
                        sv16 = sb[r, pl.ds(0, 16)]
                        for b_ in range(nb):
                            sl = pl.ds(b_ * 16, 16)
                            rows[r, sl] = rows[r, sl] * sv16
                        return _

                    lax.fori_loop(0, CHE, scale, None)
                scatter_add(rows, dstb, sc)
            return _

        lax.fori_loop(0, nk2, body2, None)
        for srcb, dstb, sb, rows, ld, g, sc in slots:
            sdrain(rows, sc)
        plsc.subcore_barrier()

        for o_, n_ in chunks:
            pltpu.sync_copy(shs[0].at[pl.ds(s * rpt + o_, n_)],
                            rows0.at[pl.ds(0, n_)])
            if nrep == 2:
                pltpu.sync_copy(shs[1].at[pl.ds(s * rpt + o_, n_)],
                                rows1.at[pl.ds(0, n_)])

                def radd(r, _):
                    for b_ in range(nb):
                        sl = pl.ds(b_ * 16, 16)
                        rows0[r, sl] = rows0[r, sl] + rows1[r, sl]
                    return _

                lax.fori_loop(0, n_, radd, None)
            pltpu.sync_copy(rows0.at[pl.ds(0, n_)],
                            agg.at[pl.ds(lo + s * rpt + o_, n_)])

    return body


def _c(f_hbm, src, dst, s_e, mp, d, e, nrep=1, premul=False):
    f = pl.kernel(
        _make_c_body(mp, d, e, nrep, premul),
        out_type=jax.ShapeDtypeStruct((mp, d), _F32),
        mesh=_mesh(),
        compiler_params=pltpu.CompilerParams(use_tc_tiling_on_sc=False),
        scratch_types=[
            pltpu.VMEM((CHE, d), _F32),
            pltpu.VMEM((CHE, d), _F32),
            pltpu.VMEM((CHE,), _I32),
            pltpu.VMEM((CHE,), _I32),
            pltpu.VMEM((CHE,), _I32),
            pltpu.VMEM((CHE,), _I32),
            pltpu.VMEM((CHE, 16), _F32),
            pltpu.VMEM((CHE, 16), _F32),
            pltpu.SemaphoreType.DMA,
            pltpu.SemaphoreType.DMA,
            pltpu.SemaphoreType.DMA,
            pltpu.SemaphoreType.DMA,
            pltpu.SemaphoreType.DMA,
            pltpu.SemaphoreType.DMA,
        ] + [pltpu.VMEM_SHARED((mp // 2 + 16, d), _F32)] * nrep,
    )
    return f(f_hbm, src, dst, s_e)


def _bn_relu(x, g, b):
    m = jnp.mean(x, axis=0)
    v = jnp.mean((x - m) ** 2, axis=0)
    return jnp.maximum((x - m) / jnp.sqrt(v + 1e-5) * g + b, 0.0)


def _make_tcb_body(inv_radius, premul_a):
    def body(gs_ref, gd_ref, kaa_ref, kba_ref, kab_ref, kbb_ref,
             sa_ref, sb_ref):
        gs = gs_ref[...]
        gd = gd_ref[...]
        rel = (gd[:, 0:3] - gs[:, 0:3]) * inv_radius
        wa = jnp.dot(jnp.maximum(jnp.dot(rel, kaa_ref[...],
                                         preferred_element_type=_F32), 0.0),
                     kba_ref[...], preferred_element_type=_F32)
        wb = jnp.dot(jnp.maximum(jnp.dot(rel, kab_ref[...],
                                         preferred_element_type=_F32), 0.0),
                     kbb_ref[...], preferred_element_type=_F32)
        invp = 1.0 / jnp.maximum(gs[:, 7:8] * (1.0 / 32.0), 1e-3)
        invd = 1.0 / jnp.maximum(gd[:, 8:9], 1.0)
        sc = invp * invd
        blk = gs.shape[0]
        if premul_a:
            sa_ref[...] = gs * jnp.broadcast_to(wa * sc, (blk, 16))
        else:
            sa_ref[...] = jnp.broadcast_to(wa * sc, (blk, 16))
        sb_ref[...] = jnp.broadcast_to(wb * sc, (blk, 16))

    return body


_TCB_BLK = 8192


def _tcb(gs, gd, kaa, kba, kab, kbb, e, inv_radius, premul_a=False):
    nblk = e // _TCB_BLK
    out = pl.pallas_call(
        _make_tcb_body(inv_radius, premul_a),
        grid=(nblk,),
        in_specs=[
            pl.BlockSpec((_TCB_BLK, 16), lambda i: (i, 0)),
            pl.BlockSpec((_TCB_BLK, 16), lambda i: (i, 0)),
            pl.BlockSpec((3, 16), lambda i: (0, 0)),
            pl.BlockSpec((16, 1), lambda i: (0, 0)),
            pl.BlockSpec((3, 16), lambda i: (0, 0)),
            pl.BlockSpec((16, 1), lambda i: (0, 0)),
        ],
        out_specs=[
            pl.BlockSpec((_TCB_BLK, 16), lambda i: (i, 0)),
            pl.BlockSpec((_TCB_BLK, 16), lambda i: (i, 0)),
        ],
        out_shape=[
            jax.ShapeDtypeStruct((e, 16), _F32),
            jax.ShapeDtypeStruct((e, 16), _F32),
        ],
    )(gs, gd, kaa, kba, kab, kbb)
    return out[0], out[1]


def _tc1_body(agg_ref, w_ref, g_ref, b_ref, out_ref):
    a = agg_ref[...]
    x = jnp.dot(a[:M1, 3:6], w_ref[...], preferred_element_type=_F32)
    out_ref[...] = _bn_relu(x, g_ref[...], b_ref[...])


def _tc2_body(agg_ref, w_ref, out_ref):
    a = agg_ref[...]
    y = jnp.dot(a[:M1], w_ref[...], preferred_element_type=_F32)
    out_ref[...] = jnp.concatenate(
        [y, jnp.zeros((M1P - M1, 128), _F32)], axis=0)


def _tc3_body(fsum_ref, cnt_ref, g_ref, b_ref, out_ref):
    f2 = fsum_ref[:M2] / jnp.maximum(cnt_ref[0, :M2], 1.0)[:, None]
    out_ref[...] = _bn_relu(f2, g_ref[...], b_ref[...])


def _tc4_body(agg_ref, w_ref, g_ref, b_ref, out_ref):
    a = agg_ref[...]
    x = jnp.dot(a[:M2], w_ref[...], preferred_element_type=_F32)
    out_ref[...] = _bn_relu(x, g_ref[...], b_ref[...])


def _tc5_body(agg_ref, w_ref, ids_ref, g4_ref, b4_ref, w1_ref, c1_ref,
              g5_ref, b5_ref, w2_ref, c2_ref, out_ref):
    a = agg_ref[...]
    f4 = jnp.dot(a[:M2], w_ref[...], preferred_element_type=_F32)
    seg = lax.broadcasted_iota(_I32, (NB, M2), 0)
    onehot = (ids_ref[...] == seg).astype(_F32)
    cnt = jnp.sum(onehot, axis=1, keepdims=True)
    gm = jnp.dot(onehot, f4, preferred_element_type=_F32)
    gm = gm / jnp.maximum(cnt, 1.0)
    gm = _bn_relu(gm, g4_ref[...], b4_ref[...])
    gm = jnp.dot(gm, w1_ref[...], preferred_element_type=_F32) + c1_ref[...]
    gm = _bn_relu(gm, g5_ref[...], b5_ref[...])
    out_ref[...] = (jnp.dot(gm, w2_ref[...], preferred_element_type=_F32)
                    + c2_ref[...])


def _tc(body, out_shape, *args):
    return pl.pallas_call(
        body, out_shape=jax.ShapeDtypeStruct(out_shape, _F32))(*args)


def kernel(pts, feats, pool_ids1, edge_index1, pool_ids2, edge_index2,
           batch_ids_out,
           k1a_11, k1b_11, w_11, bn_g1, bn_b1, k1a_12, k1b_12, w_12, bn_g2,
           bn_b2, k1a_21, k1b_21, w_21, bn_g3, bn_b3, k1a_22, k1b_22, w_22,
           bn_g4, bn_b4, fc1_w, fc1_b, bn_g5, bn_b5, fc2_w, fc2_b):
    ones_col = jnp.ones((N, 1), _F32)
    rows16 = jnp.concatenate(
        [pts, feats, ones_col, jnp.zeros((N, 9), _F32)], axis=1)
    rows16 = jnp.concatenate(
        [rows16, jnp.zeros((NP - N, 16), _F32)], axis=0)
    ids1 = jnp.concatenate(
        [pool_ids1.astype(_I32), jnp.zeros((NP - N,), _I32)])
    pe1 = jnp.full((E1P - E1,), M1, _I32)
    src1 = jnp.concatenate([edge_index1[0].astype(_I32), pe1])
    dst1 = jnp.concatenate([edge_index1[1].astype(_I32), pe1])
    pe2 = jnp.full((E2P - E2,), M2, _I32)
    src2 = jnp.concatenate([edge_index2[0].astype(_I32), pe2])
    dst2 = jnp.concatenate([edge_index2[1].astype(_I32), pe2])
    pids2 = jnp.concatenate(
        [pool_ids2.astype(_I32),
         jnp.full((M1P - M1,), M2P - 1, _I32)])
    pad0 = jnp.zeros((NP - N,), _F32)
    pxc = jnp.concatenate([pts[:, 0], pad0])
    pyc = jnp.concatenate([pts[:, 1], pad0])
    pzc = jnp.concatenate([pts[:, 2], pad0])
    onec = jnp.concatenate([jnp.ones((N,), _F32), pad0])

    table16, px1, py1, pz1 = _a1(
        rows16, ids1, src1, dst1, pxc, pyc, pzc, onec)
    gs1, gd1 = _g(table16, src1, dst1, e=E1P)
    s11, s12 = _tcb(gs1, gd1, k1a_11, k1b_11, k1a_12, k1b_12,
                    e=E1P, inv_radius=10.0, premul_a=True)
    agg11 = _c(s11, src1, dst1, s11, mp=M1P, d=16, e=E1P, premul=True)
    f1 = _tc(_tc1_body, (M1, 128), agg11, w_11, bn_g1, bn_b1)
    agg12 = _c(f1, src1, dst1, s12, mp=M1P, d=128, e=E1P)
    f12p = _tc(_tc2_body, (M1P, 128), agg12, w_12)

    fsum2, pone, table2 = _a2(f12p, pids2, px1, py1, pz1, src2, dst2)
    f2 = _tc(_tc3_body, (M2, 128), fsum2, pone.reshape(1, M2P),
             bn_g2, bn_b2)

    gs2, gd2 = _g(table2, src2, dst2, e=E2P)
    s21, s22 = _tcb(gs2, gd2, k1a_21, k1b_21, k1a_22, k1b_22,
                    e=E2P, inv_radius=5.0)
    agg21 = _c(f2, src2, dst2, s21, mp=M2P, d=128, e=E2P, nrep=2)
    f3 = _tc(_tc4_body, (M2, 128), agg21, w_21, bn_g3, bn_b3)
    agg22 = _c(f3, src2, dst2, s22, mp=M2P, d=128, e=E2P)

    ids_out = batch_ids_out.astype(_I32).reshape(1, M2)
    return _tc(_tc5_body, (NB, 40), agg22, w_22, ids_out, bn_g4, bn_b4,
               fc1_w, fc1_b, bn_g5, bn_b5, fc2_w, fc2_b)
